# Initial kernel scaffold; baseline (speedup 1.0000x reference)
import jax
import jax.numpy as jnp
from jax.experimental import pallas as pl


def kernel(table, idx):
    raise NotImplementedError("write your pallas kernel here")


# speedup vs baseline: 1.0045x; 1.0045x over previous
import math

import jax
import jax.numpy as jnp
from jax.experimental import pallas as pl
from jax.experimental.pallas import tpu as pltpu


def _cdiv(a: int, b: int) -> int:
    return -(-a // b)


def _make_gather_body(R: int, n_chunks: int):
    def _body(idx_ref, t_hbm, o_hbm, sems):
        c = pl.program_id(0)
        j = pl.program_id(1)
        bank = j & 1
        base = (c * n_chunks + j) * R

        for r in range(R):
            row = base + r
            pltpu.make_async_copy(
                t_hbm.at[idx_ref[row]],
                o_hbm.at[row],
                sems.at[bank],
            ).start()

        @pl.when(j > 0)
        def _():
            pltpu.make_async_copy(
                t_hbm.at[pl.ds(0, R)], o_hbm.at[pl.ds(0, R)], sems.at[1 - bank]
            ).wait()

        @pl.when(j == n_chunks - 1)
        def _():
            pltpu.make_async_copy(
                t_hbm.at[pl.ds(0, R)], o_hbm.at[pl.ds(0, R)], sems.at[bank]
            ).wait()

    return _body


def _gather_rows(table: jax.Array, idx: jax.Array, M: int, R: int = 128):
    row_shape = tuple(int(d) for d in table.shape[1:])
    row_bytes = max(1, math.prod(row_shape) * jnp.dtype(table.dtype).itemsize)

    NC = 2
    R = max(1, min(R, _cdiv(M, NC)))
    n_chunks = _cdiv(M, NC * R)
    M_pad = NC * n_chunks * R
    if M_pad != M:
        idx = jnp.pad(idx, (0, M_pad - M))

    grid_spec = pltpu.PrefetchScalarGridSpec(
        num_scalar_prefetch=1,
        grid=(NC, n_chunks),
        in_specs=[pl.BlockSpec(memory_space=pl.ANY)],
        out_specs=pl.BlockSpec(memory_space=pl.ANY),
        scratch_shapes=[pltpu.SemaphoreType.DMA((2,))],
    )
    cost = pl.CostEstimate(
        flops=0, transcendentals=0,
        bytes_accessed=2 * M_pad * row_bytes + 4 * M_pad,
    )
    out = pl.pallas_call(
        _make_gather_body(R, n_chunks),
        grid_spec=grid_spec,
        out_shape=jax.ShapeDtypeStruct((M_pad,) + row_shape, table.dtype),
        compiler_params=pltpu.CompilerParams(
            dimension_semantics=("parallel", "arbitrary"),
            disable_bounds_checks=True,
        ),
        cost_estimate=cost,
    )(idx, table)
    if M_pad != M:
        out = out[:M]
    return out


def kernel(table: jax.Array, idx: jax.Array) -> jax.Array:
    M = int(idx.shape[0])
    row_shape = tuple(int(d) for d in table.shape[1:])
    idx = idx.astype(jnp.int32)
    if M == 0:
        return jnp.zeros((0,) + row_shape, table.dtype)
    return _gather_rows(table, idx, M)


# speedup vs baseline: 4.3413x; 4.3218x over previous
import math

import jax
import jax.numpy as jnp
from jax.experimental import pallas as pl
from jax.experimental.pallas import tpu as pltpu


def _cdiv(a: int, b: int) -> int:
    return -(-a // b)


def _make_gather_body(R: int, n_chunks: int):
    def _body(idx_ref, t_hbm, o_ref, sem):
        c = pl.program_id(0)
        j = pl.program_id(1)
        base = (c * n_chunks + j) * R

        for r in range(R):
            pltpu.make_async_copy(
                t_hbm.at[idx_ref[base + r]],
                o_ref.at[r],
                sem,
            ).start()

        pltpu.make_async_copy(t_hbm.at[pl.ds(0, R)], o_ref, sem).wait()

    return _body


def _gather_rows(table: jax.Array, idx: jax.Array, M: int, R: int = 256):
    row_shape = tuple(int(d) for d in table.shape[1:])
    row_bytes = max(1, math.prod(row_shape) * jnp.dtype(table.dtype).itemsize)
    nd = table.ndim

    NC = 2
    R = max(1, min(R, _cdiv(M, NC)))
    n_chunks = _cdiv(M, NC * R)
    M_pad = NC * n_chunks * R
    if M_pad != M:
        idx = jnp.pad(idx, (0, M_pad - M))

    grid_spec = pltpu.PrefetchScalarGridSpec(
        num_scalar_prefetch=1,
        grid=(NC, n_chunks),
        in_specs=[pl.BlockSpec(memory_space=pl.ANY)],
        out_specs=pl.BlockSpec(
            (R,) + row_shape,
            lambda c, j, idx_ref: (c * n_chunks + j,) + (0,) * (nd - 1),
        ),
        scratch_shapes=[pltpu.SemaphoreType.DMA],
    )
    cost = pl.CostEstimate(
        flops=0, transcendentals=0,
        bytes_accessed=2 * M_pad * row_bytes + 4 * M_pad,
    )
    out = pl.pallas_call(
        _make_gather_body(R, n_chunks),
        grid_spec=grid_spec,
        out_shape=jax.ShapeDtypeStruct((M_pad,) + row_shape, table.dtype),
        compiler_params=pltpu.CompilerParams(
            dimension_semantics=("parallel", "arbitrary"),
            disable_bounds_checks=True,
        ),
        cost_estimate=cost,
    )(idx, table)
    if M_pad != M:
        out = out[:M]
    return out


def kernel(table: jax.Array, idx: jax.Array) -> jax.Array:
    M = int(idx.shape[0])
    row_shape = tuple(int(d) for d in table.shape[1:])
    idx = idx.astype(jnp.int32)
    if M == 0:
        return jnp.zeros((0,) + row_shape, table.dtype)
    return _gather_rows(table, idx, M)


# speedup vs baseline: 4.7106x; 1.0851x over previous
import math

import jax
import jax.numpy as jnp
from jax.experimental import pallas as pl
from jax.experimental.pallas import tpu as pltpu


def _cdiv(a: int, b: int) -> int:
    return -(-a // b)


def _make_gather_body(R: int, n_chunks: int):
    def _body(idx_ref, t_hbm, o_hbm, scr, rsems, wsems):
        c = pl.program_id(0)
        j = pl.program_id(1)
        bank = j & 1
        base = (c * n_chunks + j) * R

        def block_write(b, chunk_base):
            return pltpu.make_async_copy(
                scr.at[b], o_hbm.at[pl.ds(chunk_base, R)], wsems.at[b]
            )

        def read_wait(b):
            pltpu.make_async_copy(t_hbm.at[pl.ds(0, R)], scr.at[b], rsems.at[b]).wait()

        @pl.when(j > 1)
        def _():
            block_write(bank, 0).wait()

        for r in range(R):
            pltpu.make_async_copy(
                t_hbm.at[idx_ref[base + r]],
                scr.at[bank, r],
                rsems.at[bank],
            ).start()

        @pl.when(j > 0)
        def _():
            read_wait(1 - bank)
            block_write(1 - bank, base - R).start()

        @pl.when(j == n_chunks - 1)
        def _():
            read_wait(bank)
            block_write(bank, base).start()
            block_write(bank, 0).wait()
            @pl.when(j > 0)
            def _():
                block_write(1 - bank, 0).wait()

    return _body


def _gather_rows(table: jax.Array, idx: jax.Array, M: int, R: int = 256):
    row_shape = tuple(int(d) for d in table.shape[1:])
    row_bytes = max(1, math.prod(row_shape) * jnp.dtype(table.dtype).itemsize)

    NC = 2
    R = max(1, min(R, _cdiv(M, NC)))
    n_chunks = _cdiv(M, NC * R)
    M_pad = NC * n_chunks * R
    if M_pad != M:
        idx = jnp.pad(idx, (0, M_pad - M))

    grid_spec = pltpu.PrefetchScalarGridSpec(
        num_scalar_prefetch=1,
        grid=(NC, n_chunks),
        in_specs=[pl.BlockSpec(memory_space=pl.ANY)],
        out_specs=pl.BlockSpec(memory_space=pl.ANY),
        scratch_shapes=[
            pltpu.VMEM((2, R) + row_shape, table.dtype),
            pltpu.SemaphoreType.DMA((2,)),
            pltpu.SemaphoreType.DMA((2,)),
        ],
    )
    cost = pl.CostEstimate(
        flops=0, transcendentals=0,
        bytes_accessed=2 * M_pad * row_bytes + 4 * M_pad,
    )
    out = pl.pallas_call(
        _make_gather_body(R, n_chunks),
        grid_spec=grid_spec,
        out_shape=jax.ShapeDtypeStruct((M_pad,) + row_shape, table.dtype),
        compiler_params=pltpu.CompilerParams(
            dimension_semantics=("parallel", "arbitrary"),
            disable_bounds_checks=True,
        ),
        cost_estimate=cost,
    )(idx, table)
    if M_pad != M:
        out = out[:M]
    return out


def kernel(table: jax.Array, idx: jax.Array) -> jax.Array:
    M = int(idx.shape[0])
    row_shape = tuple(int(d) for d in table.shape[1:])
    idx = idx.astype(jnp.int32)
    if M == 0:
        return jnp.zeros((0,) + row_shape, table.dtype)
    return _gather_rows(table, idx, M)
